# baseline (speedup 1.0000x reference)
import jax
import jax.numpy as jnp
from jax import lax
from jax.experimental import pallas as pl
from jax.experimental.pallas import tpu as pltpu
from jax.experimental.pallas import tpu_sc as plsc

_NC = 2
_NS = 16
_NW = _NC * _NS

_ROWS, _COLS = 4096, 200
_B = _ROWS * _COLS
_D = 64
_DP = 128
_B_PER_W = _B // _NW
_G = 128
_CHUNK_G = 4
_CHUNK = _CHUNK_G * _G
_N_CHUNKS = _B_PER_W // _CHUNK
_NBUF = 2


def _gather_body(idx_hbm, tab_hbm, out_hbm, idx_v, idx2_v, rows_v,
                 isem0, isem1, gsem0, gsem1, wsem0, wsem1):
    isem = [isem0, isem1]
    gsem = [gsem0, gsem1]
    wsem = [wsem0, wsem1]
    wid = lax.axis_index("s") * _NC + lax.axis_index("c")
    base = wid * _B_PER_W
    row_base = base // _G

    def idx_copy(c, b):
        row0 = pl.multiple_of(row_base + c * _CHUNK_G, _CHUNK_G)
        return pltpu.make_async_copy(
            idx_hbm.at[pl.ds(row0, _CHUNK_G)], idx_v.at[b], isem[b])

    def gather_copies(b):
        return [
            pltpu.make_async_copy(
                tab_hbm.at[idx2_v.at[b].at[j]],
                rows_v.at[b].at[pl.ds(j * _G, _G)],
                gsem[b],
            )
            for j in range(_CHUNK_G)
        ]

    def wb_copy(c, b):
        start = pl.multiple_of(base + c * _CHUNK, _CHUNK)
        return pltpu.make_async_copy(
            rows_v.at[b],
            out_hbm.at[pl.ds(start, _CHUNK), pl.ds(0, _D)],
            wsem[b])

    for b in range(_NBUF):
        idx_copy(b, b).start()

    @pl.loop(0, _N_CHUNKS, step=_NBUF)
    def _super(g):
        for b in range(_NBUF):
            c = g + b
            b1 = 1 - b
            idx_copy(c, b).wait()
            for j in range(_CHUNK_G):
                for v in range(_G // 16):
                    s = pl.ds(v * 16, 16)
                    idx2_v[b, j, s] = idx_v[b, j, s] * 2
            @pl.when(c >= _NBUF)
            def _():
                wb_copy(c, b).wait()
            for cp in gather_copies(b):
                cp.start()
            @pl.when(c + _NBUF < _N_CHUNKS)
            def _():
                idx_copy(c + _NBUF, b).start()
            @pl.when(c >= 1)
            def _():
                for cp in gather_copies(b1):
                    cp.wait()
                wb_copy(c - 1, b1).start()

    last_b = (_N_CHUNKS - 1) % _NBUF
    for cp in gather_copies(last_b):
        cp.wait()
    wb_copy(_N_CHUNKS - 1, last_b).start()
    for b in range(_NBUF):
        wb_copy(_N_CHUNKS - _NBUF + b, b).wait()


_mesh = plsc.VectorSubcoreMesh(core_axis_name="c", subcore_axis_name="s")

_gather = pl.kernel(
    _gather_body,
    out_type=jax.ShapeDtypeStruct((_B, _DP), jnp.float32),
    mesh=_mesh,
    compiler_params=pltpu.CompilerParams(
        use_tc_tiling_on_sc=False, needs_layout_passes=False),
    scratch_types=[
        pltpu.VMEM((_NBUF, _CHUNK_G, _G), jnp.int32),
        pltpu.VMEM((_NBUF, _CHUNK_G, _G), jnp.int32),
        pltpu.VMEM((_NBUF, _CHUNK, _D), jnp.float32),
        pltpu.SemaphoreType.DMA,
        pltpu.SemaphoreType.DMA,
        pltpu.SemaphoreType.DMA,
        pltpu.SemaphoreType.DMA,
        pltpu.SemaphoreType.DMA,
        pltpu.SemaphoreType.DMA,
    ],
)


def kernel(x, table):
    idx = x.reshape(_B // _G, _G).astype(jnp.int32)
    tab = jnp.pad(table, ((0, 0), (0, _DP - _D))).reshape(2 * 1000000, _D)
    out = _gather(idx, tab)
    return out[:, :_D].reshape(_ROWS, _COLS, _D)
